# baseline (speedup 1.0000x reference)
import dataclasses
import functools

import jax
import jax.numpy as jnp
from jax import lax
from jax.experimental import pallas as pl
from jax.experimental.pallas import tpu as pltpu
from jax.experimental.pallas import tpu_sc as plsc

N = 10000
E = 320000
C = 128

NW = 32
R = 320
NPAD = NW * R
CHUNK = 1280
GB = 32
CAP = 10752
TH = CAP - CHUNK
L = 16


def _tc_tables(xg, xr, W, b):
    BLK = 1024
    dn = (((1,), (1,)), ((), ()))

    def body(xg_ref, xr_ref, w_ref, b_ref, g_ref, rb_ref):
        w = w_ref[...]
        g_ref[...] = lax.dot_general(xg_ref[...], w, dn,
                                     preferred_element_type=jnp.float32)
        rb_ref[...] = lax.dot_general(xr_ref[...], w, dn,
                                      preferred_element_type=jnp.float32) - b_ref[...]

    return pl.pallas_call(
        body,
        grid=(NPAD // BLK,),
        in_specs=[
            pl.BlockSpec((BLK, C), lambda i: (i, 0)),
            pl.BlockSpec((BLK, C), lambda i: (i, 0)),
            pl.BlockSpec((C, C), lambda i: (0, 0)),
            pl.BlockSpec((1, C), lambda i: (0, 0)),
        ],
        out_specs=[
            pl.BlockSpec((BLK, C), lambda i: (i, 0)),
            pl.BlockSpec((BLK, C), lambda i: (i, 0)),
        ],
        out_shape=[jax.ShapeDtypeStruct((NPAD, C), jnp.float32)] * 2,
    )(xg, xr, W, b.reshape(1, C))


def _sc_segment_reduce(src, dst, t):
    mesh = plsc.VectorSubcoreMesh(core_axis_name="c", subcore_axis_name="s")
    cp = pltpu.CompilerParams()
    if "needs_layout_passes" in pltpu.CompilerParams.__dataclass_fields__:
        cp = dataclasses.replace(cp, needs_layout_passes=False)

    @functools.partial(
        pl.kernel,
        out_type=[jax.ShapeDtypeStruct((NPAD * C,), jnp.float32)] * 2,
        mesh=mesh,
        compiler_params=cp,
        scratch_types=[
            pltpu.VMEM((R * C,), jnp.float32),
            pltpu.VMEM((R * C,), jnp.float32),
            pltpu.VMEM((R * L,), jnp.float32),
            pltpu.VMEM((CHUNK,), jnp.int32),
            pltpu.VMEM((CHUNK,), jnp.int32),
            pltpu.VMEM((CHUNK,), jnp.int32),
            pltpu.VMEM((CHUNK,), jnp.int32),
            pltpu.VMEM((2 * CAP,), jnp.int32),
            pltpu.VMEM((2 * GB, C), jnp.float32),
            pltpu.VMEM((2 * GB, C), jnp.float32),
            pltpu.SemaphoreType.DMA,
            pltpu.SemaphoreType.DMA,
            pltpu.SemaphoreType.DMA,
            pltpu.SemaphoreType.DMA,
        ],
    )
    def sc_kernel(src_hbm, dst_hbm, t_hbm, omax_hbm, omean_hbm,
                  acc_max, acc_sum, cnt, sbufa, dbufa, sbufb, dbufb,
                  ilist, ts0, ts1,
                  semca, semcb, semg0, semg1):
        wid = lax.axis_index("s") * 2 + lax.axis_index("c")
        lo = wid * R
        zvec = jnp.zeros((L,), jnp.float32)
        zivec = jnp.zeros((L,), jnp.int32)
        ones = jnp.ones((L,), jnp.float32)
        lanes = lax.iota(jnp.int32, L)

        @pl.loop(0, R * C, step=L)
        def _init(i):
            acc_max[pl.ds(i, L)] = zvec
            acc_sum[pl.ds(i, L)] = zvec

        @pl.loop(0, R * L, step=L)
        def _init_cnt(i):
            cnt[pl.ds(i, L)] = zvec

        @pl.loop(0, 2 * CAP, step=L)
        def _init_lists(i):
            ilist[pl.ds(i, L)] = zivec

        def fire_gather(bi, ts, sg):
            b0 = 2 * bi * GB
            pltpu.make_async_copy(t_hbm.at[ilist.at[pl.ds(b0, 2 * GB)]],
                                  ts, sg).start()

        def wait_gather(bi, ts, sg):
            b0 = 2 * bi * GB
            pltpu.make_async_copy(t_hbm.at[ilist.at[pl.ds(b0, 2 * GB)]],
                                  ts, sg).wait()

        def accum(b0, nb, ts):
            def edge(j, _):
                pos = b0 + j
                dvj = plsc.load_gather(
                    ilist, [jnp.full((L,), 2 * pos + 1, jnp.int32)])
                rowbase = (dvj - (NPAD + lo)) * C + lanes
                for c in range(0, C, L):
                    gv = ts[2 * j, pl.ds(c, L)]
                    rv = ts[2 * j + 1, pl.ds(c, L)]
                    z = jnp.maximum(gv - rv, 0.0)
                    idxv = rowbase + c
                    mold = plsc.load_gather(acc_max, [idxv])
                    plsc.store_scatter(acc_max, [idxv], jnp.maximum(mold, z))
                    plsc.addupdate_scatter(acc_sum, [idxv], z)
                plsc.addupdate_scatter(cnt, [(dvj - (NPAD + lo)) * L + lanes],
                                       ones)
                return 0

            lax.fori_loop(0, nb, edge, 0)

        def drain(n):
            nbatch = (n + GB - 1) // GB
            npair = (nbatch + 1) // 2
            fire_gather(0, ts0, semg0)

            def pair_body(p, _):
                b0 = 2 * p * GB
                wait_gather(2 * p, ts0, semg0)
                fire_gather(2 * p + 1, ts1, semg1)
                accum(b0, jnp.maximum(0, jnp.minimum(GB, n - b0)), ts0)
                wait_gather(2 * p + 1, ts1, semg1)
                fire_gather(2 * p + 2, ts0, semg0)
                accum(b0 + GB, jnp.maximum(0, jnp.minimum(GB, n - b0 - GB)),
                      ts1)
                return 0

            lax.fori_loop(0, npair, pair_body, 0)
            wait_gather(2 * npair, ts0, semg0)

        def fire_chunk(ci, sbuf, dbuf, sem):
            e0 = ci * CHUNK
            pltpu.make_async_copy(src_hbm.at[pl.ds(e0, CHUNK)], sbuf,
                                  sem).start()
            pltpu.make_async_copy(dst_hbm.at[pl.ds(e0, CHUNK)], dbuf,
                                  sem).start()

        def wait_chunk(ci, sbuf, dbuf, sem):
            e0 = ci * CHUNK
            pltpu.make_async_copy(src_hbm.at[pl.ds(e0, CHUNK)], sbuf,
                                  sem).wait()
            pltpu.make_async_copy(dst_hbm.at[pl.ds(e0, CHUNK)], dbuf,
                                  sem).wait()

        def scan_chunk(sbuf, dbuf, off_v):
            def body(k, ov):
                d = dbuf[pl.ds(k * L, L)]
                s = sbuf[pl.ds(k * L, L)]
                ui = d - lo
                m = plsc.bitcast(ui, jnp.uint32) < jnp.uint32(R)
                mi = jnp.where(m, 1, 0)
                pos2 = (ov + plsc.cumsum(mi) - mi) * 2
                plsc.store_scatter(ilist, [pos2], s, mask=m)
                plsc.store_scatter(ilist, [pos2 + 1], d + NPAD, mask=m)
                return ov + plsc.all_reduce_population_count(m)

            return lax.fori_loop(0, CHUNK // L, body, off_v, unroll=4)

        def maybe_drain(off_v):
            off = jnp.max(off_v)
            cond = off > TH

            @pl.when(cond)
            def _():
                drain(off)

            return jnp.where(cond, jnp.zeros((L,), jnp.int32), off_v)

        NCH = E // CHUNK
        fire_chunk(0, sbufa, dbufa, semca)

        def chunk_pair(t, off_v):
            ci = 2 * t
            wait_chunk(ci, sbufa, dbufa, semca)
            fire_chunk(ci + 1, sbufb, dbufb, semcb)
            off_v = scan_chunk(sbufa, dbufa, off_v)
            off_v = maybe_drain(off_v)
            wait_chunk(ci + 1, sbufb, dbufb, semcb)
            fire_chunk(jnp.minimum(ci + 2, NCH - 2), sbufa, dbufa, semca)
            off_v = scan_chunk(sbufb, dbufb, off_v)
            off_v = maybe_drain(off_v)
            return off_v

        off_v = lax.fori_loop(0, NCH // 2, chunk_pair,
                              jnp.zeros((L,), jnp.int32))
        wait_chunk(NCH - 2, sbufa, dbufa, semca)
        drain(jnp.max(off_v))

        @pl.loop(0, R)
        def _finalize(r):
            denom = jnp.maximum(cnt[pl.ds(r * L, L)], 1.0)
            for c in range(0, C, L):
                slc = acc_sum.at[pl.ds(r * C + c, L)]
                slc[...] = slc[...] / denom

        pltpu.sync_copy(acc_max, omax_hbm.at[pl.ds(lo * C, R * C)])
        pltpu.sync_copy(acc_sum, omean_hbm.at[pl.ds(lo * C, R * C)])

    return sc_kernel(src, dst, t)


def kernel(x_g, x_rsc, edge_index, W, b):
    src = edge_index[0]
    dst = edge_index[1]
    pad = ((0, NPAD - N), (0, 0))
    xg = jnp.pad(x_g, pad)
    xr = jnp.pad(x_rsc, pad)
    g, rb = _tc_tables(xg, xr, W, b)
    t = jnp.concatenate([g, rb], axis=0)
    omax, omean = _sc_segment_reduce(src, dst, t)
    omax = omax.reshape(NPAD, C)[:N]
    omean = omean.reshape(NPAD, C)[:N]
    return jnp.concatenate([omax, omean], axis=-1)


# speedup vs baseline: 1.1299x; 1.1299x over previous
import dataclasses
import functools

import jax
import jax.numpy as jnp
from jax import lax
from jax.experimental import pallas as pl
from jax.experimental.pallas import tpu as pltpu
from jax.experimental.pallas import tpu_sc as plsc

N = 10000
E = 320000
C = 128

NW = 32
R = 320
NPAD = NW * R
CHUNK = 1280
GB = 32
CAP = 8192
MASK = CAP - 1
TH = CAP - CHUNK
L = 16


def _tc_tables(xg, xr, W, b):
    BLK = 1024
    dn = (((1,), (1,)), ((), ()))

    def body(xg_ref, xr_ref, w_ref, b_ref, g_ref, rb_ref):
        w = w_ref[...]
        g_ref[...] = lax.dot_general(xg_ref[...], w, dn,
                                     preferred_element_type=jnp.float32)
        rb_ref[...] = lax.dot_general(xr_ref[...], w, dn,
                                      preferred_element_type=jnp.float32) - b_ref[...]

    return pl.pallas_call(
        body,
        grid=(NPAD // BLK,),
        in_specs=[
            pl.BlockSpec((BLK, C), lambda i: (i, 0)),
            pl.BlockSpec((BLK, C), lambda i: (i, 0)),
            pl.BlockSpec((C, C), lambda i: (0, 0)),
            pl.BlockSpec((1, C), lambda i: (0, 0)),
        ],
        out_specs=[
            pl.BlockSpec((BLK, C), lambda i: (i, 0)),
            pl.BlockSpec((BLK, C), lambda i: (i, 0)),
        ],
        out_shape=[jax.ShapeDtypeStruct((NPAD, C), jnp.float32)] * 2,
    )(xg, xr, W, b.reshape(1, C))


def _sc_segment_reduce(src, dst, g, rb):
    mesh = plsc.VectorSubcoreMesh(core_axis_name="c", subcore_axis_name="s")
    cp = pltpu.CompilerParams()
    if "needs_layout_passes" in pltpu.CompilerParams.__dataclass_fields__:
        cp = dataclasses.replace(cp, needs_layout_passes=False)

    @functools.partial(
        pl.kernel,
        out_type=[jax.ShapeDtypeStruct((NPAD * C,), jnp.float32)] * 2,
        mesh=mesh,
        compiler_params=cp,
        scratch_types=[
            pltpu.VMEM((R * C,), jnp.float32),
            pltpu.VMEM((R * C,), jnp.float32),
            pltpu.VMEM((R * L,), jnp.float32),
            pltpu.VMEM((CHUNK,), jnp.int32),
            pltpu.VMEM((CHUNK,), jnp.int32),
            pltpu.VMEM((CHUNK,), jnp.int32),
            pltpu.VMEM((CHUNK,), jnp.int32),
            pltpu.VMEM((CAP,), jnp.int32),
            pltpu.VMEM((CAP,), jnp.int32),
            pltpu.VMEM((GB, C), jnp.float32),
            pltpu.VMEM((GB, C), jnp.float32),
            pltpu.VMEM((GB, C), jnp.float32),
            pltpu.VMEM((GB, C), jnp.float32),
            pltpu.SemaphoreType.DMA,
            pltpu.SemaphoreType.DMA,
            pltpu.SemaphoreType.DMA,
            pltpu.SemaphoreType.DMA,
            pltpu.SemaphoreType.DMA,
            pltpu.SemaphoreType.DMA,
        ],
    )
    def sc_kernel(src_hbm, dst_hbm, g_hbm, rb_hbm, omax_hbm, omean_hbm,
                  acc_max, acc_sum, cnt, sbufa, dbufa, sbufb, dbufb,
                  slist, dlist, gr0, rr0, gr1, rr1,
                  semca, semcb, semg0, semr0, semg1, semr1):
        wid = lax.axis_index("s") * 2 + lax.axis_index("c")
        lo = wid * R
        zvec = jnp.zeros((L,), jnp.float32)
        zivec = jnp.zeros((L,), jnp.int32)
        ones = jnp.ones((L,), jnp.float32)
        lanes = lax.iota(jnp.int32, L)

        @pl.loop(0, R * C, step=L)
        def _init(i):
            acc_max[pl.ds(i, L)] = zvec
            acc_sum[pl.ds(i, L)] = zvec

        @pl.loop(0, R * L, step=L)
        def _init_cnt(i):
            cnt[pl.ds(i, L)] = zvec

        @pl.loop(0, CAP, step=L)
        def _init_lists(i):
            slist[pl.ds(i, L)] = zivec
            dlist[pl.ds(i, L)] = zivec

        def fire_gather(b0, grows, rrows, sg, sr):
            cg = pltpu.make_async_copy(g_hbm.at[slist.at[pl.ds(b0, GB)]],
                                       grows, sg)
            cr = pltpu.make_async_copy(rb_hbm.at[dlist.at[pl.ds(b0, GB)]],
                                       rrows, sr)
            cg.start()
            cr.start()

        def wait_gather(b0, grows, rrows, sg, sr):
            pltpu.make_async_copy(g_hbm.at[slist.at[pl.ds(b0, GB)]],
                                  grows, sg).wait()
            pltpu.make_async_copy(rb_hbm.at[dlist.at[pl.ds(b0, GB)]],
                                  rrows, sr).wait()

        def accum(b0, nb, grows, rrows):
            def edge(j, _):
                pos = b0 + j
                dvj = plsc.load_gather(dlist, [jnp.full((L,), pos, jnp.int32)])
                rowbase = (dvj - lo) * C + lanes
                for c in range(0, C, L):
                    gv = grows[j, pl.ds(c, L)]
                    rv = rrows[j, pl.ds(c, L)]
                    z = jnp.maximum(gv - rv, 0.0)
                    idxv = rowbase + c
                    mold = plsc.load_gather(acc_max, [idxv])
                    plsc.store_scatter(acc_max, [idxv], jnp.maximum(mold, z))
                    plsc.addupdate_scatter(acc_sum, [idxv], z)
                plsc.addupdate_scatter(cnt, [(dvj - lo) * L + lanes], ones)
                return 0

            lax.fori_loop(0, nb, edge, 0)

        def fire_pair(b0):
            fire_gather(b0, gr0, rr0, semg0, semr0)
            fire_gather(b0 + GB, gr1, rr1, semg1, semr1)

        def consume_pair(b0, n0, n1):
            wait_gather(b0, gr0, rr0, semg0, semr0)
            wait_gather(b0 + GB, gr1, rr1, semg1, semr1)
            accum(b0, n0, gr0, rr0)
            accum(b0 + GB, n1, gr1, rr1)

        def drain_pairs(cons, off, floor):
            def cond(cn):
                return off - cn > floor

            def body(cn):
                b0 = pl.multiple_of(cn & MASK, GB)
                fire_pair(b0)
                consume_pair(b0, GB, GB)
                return cn + 2 * GB

            return lax.while_loop(cond, body, cons)

        def fire_chunk(ci, sbuf, dbuf, sem):
            e0 = ci * CHUNK
            pltpu.make_async_copy(src_hbm.at[pl.ds(e0, CHUNK)], sbuf,
                                  sem).start()
            pltpu.make_async_copy(dst_hbm.at[pl.ds(e0, CHUNK)], dbuf,
                                  sem).start()

        def wait_chunk(ci, sbuf, dbuf, sem):
            e0 = ci * CHUNK
            pltpu.make_async_copy(src_hbm.at[pl.ds(e0, CHUNK)], sbuf,
                                  sem).wait()
            pltpu.make_async_copy(dst_hbm.at[pl.ds(e0, CHUNK)], dbuf,
                                  sem).wait()

        def scan_chunk(sbuf, dbuf, off_v):
            def body(k, ov):
                d = dbuf[pl.ds(k * L, L)]
                s = sbuf[pl.ds(k * L, L)]
                ui = d - lo
                m = plsc.bitcast(ui, jnp.uint32) < jnp.uint32(R)
                mi = jnp.where(m, 1, 0)
                pos = (ov + plsc.cumsum(mi) - mi) & MASK
                plsc.store_scatter(slist, [pos], s, mask=m)
                plsc.store_scatter(dlist, [pos], d, mask=m)
                return ov + plsc.all_reduce_population_count(m)

            return lax.fori_loop(0, CHUNK // L, body, off_v, unroll=4)

        NCH = E // CHUNK
        fire_chunk(0, sbufa, dbufa, semca)

        def half_step(off_v, cons):
            off = jnp.max(off_v)
            do = off - cons >= 2 * GB
            b0 = pl.multiple_of(cons & MASK, GB)

            @pl.when(do)
            def _():
                fire_pair(b0)

            return off, do, b0

        def half_finish(off_v, do, b0, cons):
            @pl.when(do)
            def _():
                consume_pair(b0, GB, GB)

            cons = jnp.where(do, cons + 2 * GB, cons)
            return drain_pairs(cons, jnp.max(off_v), TH)

        def chunk_pair(t, carry):
            off_v, cons = carry
            ci = 2 * t
            wait_chunk(ci, sbufa, dbufa, semca)
            fire_chunk(ci + 1, sbufb, dbufb, semcb)
            off, do, b0 = half_step(off_v, cons)
            off_v = scan_chunk(sbufa, dbufa, off_v)
            cons = half_finish(off_v, do, b0, cons)
            wait_chunk(ci + 1, sbufb, dbufb, semcb)
            fire_chunk(jnp.minimum(ci + 2, NCH - 2), sbufa, dbufa, semca)
            off, do, b0 = half_step(off_v, cons)
            off_v = scan_chunk(sbufb, dbufb, off_v)
            cons = half_finish(off_v, do, b0, cons)
            return off_v, cons

        off_v, cons = lax.fori_loop(
            0, NCH // 2, chunk_pair,
            (jnp.zeros((L,), jnp.int32), jnp.int32(0)))
        wait_chunk(NCH - 2, sbufa, dbufa, semca)
        off = jnp.max(off_v)
        cons = drain_pairs(cons, off, 2 * GB - 1)
        pend = off - cons
        bf = pl.multiple_of(cons & MASK, GB)
        fire_pair(bf)
        consume_pair(bf, jnp.maximum(0, jnp.minimum(GB, pend)),
                     jnp.maximum(0, jnp.minimum(GB, pend - GB)))

        @pl.loop(0, R)
        def _finalize(r):
            denom = jnp.maximum(cnt[pl.ds(r * L, L)], 1.0)
            for c in range(0, C, L):
                slc = acc_sum.at[pl.ds(r * C + c, L)]
                slc[...] = slc[...] / denom

        pltpu.sync_copy(acc_max, omax_hbm.at[pl.ds(lo * C, R * C)])
        pltpu.sync_copy(acc_sum, omean_hbm.at[pl.ds(lo * C, R * C)])

    return sc_kernel(src, dst, g, rb)


def kernel(x_g, x_rsc, edge_index, W, b):
    src = edge_index[0]
    dst = edge_index[1]
    pad = ((0, NPAD - N), (0, 0))
    xg = jnp.pad(x_g, pad)
    xr = jnp.pad(x_rsc, pad)
    g, rb = _tc_tables(xg, xr, W, b)
    omax, omean = _sc_segment_reduce(src, dst, g, rb)
    omax = omax.reshape(NPAD, C)[:N]
    omean = omean.reshape(NPAD, C)[:N]
    return jnp.concatenate([omax, omean], axis=-1)


# speedup vs baseline: 1.1460x; 1.0143x over previous
import dataclasses
import functools

import jax
import jax.numpy as jnp
from jax import lax
from jax.experimental import pallas as pl
from jax.experimental.pallas import tpu as pltpu
from jax.experimental.pallas import tpu_sc as plsc

N = 10000
E = 320000
C = 128

NW = 32
R = 320
NPAD = NW * R
CHUNK = 2000
GB = 32
CAP = 8192
MASK = CAP - 1
TH = CAP - CHUNK
L = 16


def _tc_tables(xg, xr, W, b):
    BLK = 1024
    dn = (((1,), (1,)), ((), ()))

    def body(xg_ref, xr_ref, w_ref, b_ref, g_ref, rb_ref):
        w = w_ref[...]
        g_ref[...] = lax.dot_general(xg_ref[...], w, dn,
                                     preferred_element_type=jnp.float32)
        rb_ref[...] = lax.dot_general(xr_ref[...], w, dn,
                                      preferred_element_type=jnp.float32) - b_ref[...]

    return pl.pallas_call(
        body,
        grid=(NPAD // BLK,),
        in_specs=[
            pl.BlockSpec((BLK, C), lambda i: (i, 0)),
            pl.BlockSpec((BLK, C), lambda i: (i, 0)),
            pl.BlockSpec((C, C), lambda i: (0, 0)),
            pl.BlockSpec((1, C), lambda i: (0, 0)),
        ],
        out_specs=[
            pl.BlockSpec((BLK, C), lambda i: (i, 0)),
            pl.BlockSpec((BLK, C), lambda i: (i, 0)),
        ],
        out_shape=[jax.ShapeDtypeStruct((NPAD, C), jnp.float32)] * 2,
    )(xg, xr, W, b.reshape(1, C))


def _sc_segment_reduce(src, dst, g, rb):
    mesh = plsc.VectorSubcoreMesh(core_axis_name="c", subcore_axis_name="s")
    cp = pltpu.CompilerParams()
    if "needs_layout_passes" in pltpu.CompilerParams.__dataclass_fields__:
        cp = dataclasses.replace(cp, needs_layout_passes=False)

    @functools.partial(
        pl.kernel,
        out_type=[jax.ShapeDtypeStruct((NPAD * C,), jnp.float32)] * 2,
        mesh=mesh,
        compiler_params=cp,
        scratch_types=[
            pltpu.VMEM((R * C,), jnp.float32),
            pltpu.VMEM((R * C,), jnp.float32),
            pltpu.VMEM((R * L,), jnp.float32),
            pltpu.VMEM((CHUNK,), jnp.int32),
            pltpu.VMEM((CHUNK,), jnp.int32),
            pltpu.VMEM((CHUNK,), jnp.int32),
            pltpu.VMEM((CHUNK,), jnp.int32),
            pltpu.VMEM((CAP,), jnp.int32),
            pltpu.VMEM((CAP,), jnp.int32),
            pltpu.VMEM((GB, C), jnp.float32),
            pltpu.VMEM((GB, C), jnp.float32),
            pltpu.VMEM((GB, C), jnp.float32),
            pltpu.VMEM((GB, C), jnp.float32),
            pltpu.SemaphoreType.DMA,
            pltpu.SemaphoreType.DMA,
            pltpu.SemaphoreType.DMA,
            pltpu.SemaphoreType.DMA,
            pltpu.SemaphoreType.DMA,
            pltpu.SemaphoreType.DMA,
        ],
    )
    def sc_kernel(src_hbm, dst_hbm, g_hbm, rb_hbm, omax_hbm, omean_hbm,
                  acc_max, acc_sum, cnt, sbufa, dbufa, sbufb, dbufb,
                  slist, dlist, gr0, rr0, gr1, rr1,
                  semca, semcb, semg0, semr0, semg1, semr1):
        wid = lax.axis_index("s") * 2 + lax.axis_index("c")
        lo = wid * R
        zvec = jnp.zeros((L,), jnp.float32)
        zivec = jnp.zeros((L,), jnp.int32)
        ones = jnp.ones((L,), jnp.float32)
        lanes = lax.iota(jnp.int32, L)

        @pl.loop(0, R * C, step=L)
        def _init(i):
            acc_max[pl.ds(i, L)] = zvec
            acc_sum[pl.ds(i, L)] = zvec

        @pl.loop(0, R * L, step=L)
        def _init_cnt(i):
            cnt[pl.ds(i, L)] = zvec

        @pl.loop(0, CAP, step=L)
        def _init_lists(i):
            slist[pl.ds(i, L)] = zivec
            dlist[pl.ds(i, L)] = zivec

        def fire_gather(b0, grows, rrows, sg, sr):
            cg = pltpu.make_async_copy(g_hbm.at[slist.at[pl.ds(b0, GB)]],
                                       grows, sg)
            cr = pltpu.make_async_copy(rb_hbm.at[dlist.at[pl.ds(b0, GB)]],
                                       rrows, sr)
            cg.start()
            cr.start()

        def wait_gather(b0, grows, rrows, sg, sr):
            pltpu.make_async_copy(g_hbm.at[slist.at[pl.ds(b0, GB)]],
                                  grows, sg).wait()
            pltpu.make_async_copy(rb_hbm.at[dlist.at[pl.ds(b0, GB)]],
                                  rrows, sr).wait()

        def accum(b0, nb, grows, rrows):
            def edge(j, _):
                pos = b0 + j
                dvj = plsc.load_gather(dlist, [jnp.full((L,), pos, jnp.int32)])
                rowbase = (dvj - lo) * C + lanes
                for c in range(0, C, L):
                    gv = grows[j, pl.ds(c, L)]
                    rv = rrows[j, pl.ds(c, L)]
                    z = jnp.maximum(gv - rv, 0.0)
                    idxv = rowbase + c
                    mold = plsc.load_gather(acc_max, [idxv])
                    plsc.store_scatter(acc_max, [idxv], jnp.maximum(mold, z))
                    plsc.addupdate_scatter(acc_sum, [idxv], z)
                plsc.addupdate_scatter(cnt, [(dvj - lo) * L + lanes], ones)
                return 0

            lax.fori_loop(0, nb, edge, 0,
                          unroll=2 if isinstance(nb, int) else 1)

        def fire_pair(b0):
            fire_gather(b0, gr0, rr0, semg0, semr0)
            fire_gather(b0 + GB, gr1, rr1, semg1, semr1)

        def consume_pair(b0, n0, n1):
            wait_gather(b0, gr0, rr0, semg0, semr0)
            wait_gather(b0 + GB, gr1, rr1, semg1, semr1)
            accum(b0, n0, gr0, rr0)
            accum(b0 + GB, n1, gr1, rr1)

        def drain_pairs(cons, off, floor):
            def cond(cn):
                return off - cn > floor

            def body(cn):
                b0 = pl.multiple_of(cn & MASK, GB)
                fire_pair(b0)
                consume_pair(b0, GB, GB)
                return cn + 2 * GB

            return lax.while_loop(cond, body, cons)

        def fire_chunk(ci, sbuf, dbuf, sem):
            e0 = ci * CHUNK
            pltpu.make_async_copy(src_hbm.at[pl.ds(e0, CHUNK)], sbuf,
                                  sem).start()
            pltpu.make_async_copy(dst_hbm.at[pl.ds(e0, CHUNK)], dbuf,
                                  sem).start()

        def wait_chunk(ci, sbuf, dbuf, sem):
            e0 = ci * CHUNK
            pltpu.make_async_copy(src_hbm.at[pl.ds(e0, CHUNK)], sbuf,
                                  sem).wait()
            pltpu.make_async_copy(dst_hbm.at[pl.ds(e0, CHUNK)], dbuf,
                                  sem).wait()

        def scan_chunk(sbuf, dbuf, off_v):
            def body(k, ov):
                d = dbuf[pl.ds(k * L, L)]
                s = sbuf[pl.ds(k * L, L)]
                ui = d - lo
                m = plsc.bitcast(ui, jnp.uint32) < jnp.uint32(R)
                mi = jnp.where(m, 1, 0)
                pos = (ov + plsc.cumsum(mi) - mi) & MASK
                plsc.store_scatter(slist, [pos], s, mask=m)
                plsc.store_scatter(dlist, [pos], d, mask=m)
                return ov + plsc.all_reduce_population_count(m)

            return lax.fori_loop(0, CHUNK // L, body, off_v, unroll=4)

        NCH = E // CHUNK
        fire_chunk(0, sbufa, dbufa, semca)

        def half_step(off_v, cons):
            off = jnp.max(off_v)
            do = off - cons >= 2 * GB
            b0 = pl.multiple_of(cons & MASK, GB)

            @pl.when(do)
            def _():
                fire_pair(b0)

            return off, do, b0

        def half_finish(off_v, do, b0, cons):
            @pl.when(do)
            def _():
                consume_pair(b0, GB, GB)

            cons = jnp.where(do, cons + 2 * GB, cons)
            return drain_pairs(cons, jnp.max(off_v), TH)

        def chunk_pair(t, carry):
            off_v, cons = carry
            ci = 2 * t
            wait_chunk(ci, sbufa, dbufa, semca)
            fire_chunk(ci + 1, sbufb, dbufb, semcb)
            off, do, b0 = half_step(off_v, cons)
            off_v = scan_chunk(sbufa, dbufa, off_v)
            cons = half_finish(off_v, do, b0, cons)
            wait_chunk(ci + 1, sbufb, dbufb, semcb)
            fire_chunk(jnp.minimum(ci + 2, NCH - 2), sbufa, dbufa, semca)
            off, do, b0 = half_step(off_v, cons)
            off_v = scan_chunk(sbufb, dbufb, off_v)
            cons = half_finish(off_v, do, b0, cons)
            return off_v, cons

        off_v, cons = lax.fori_loop(
            0, NCH // 2, chunk_pair,
            (jnp.zeros((L,), jnp.int32), jnp.int32(0)))
        wait_chunk(NCH - 2, sbufa, dbufa, semca)
        off = jnp.max(off_v)
        cons = drain_pairs(cons, off, 2 * GB - 1)
        pend = off - cons
        bf = pl.multiple_of(cons & MASK, GB)
        fire_pair(bf)
        consume_pair(bf, jnp.maximum(0, jnp.minimum(GB, pend)),
                     jnp.maximum(0, jnp.minimum(GB, pend - GB)))

        @pl.loop(0, R)
        def _finalize(r):
            denom = jnp.maximum(cnt[pl.ds(r * L, L)], 1.0)
            for c in range(0, C, L):
                slc = acc_sum.at[pl.ds(r * C + c, L)]
                slc[...] = slc[...] / denom

        pltpu.sync_copy(acc_max, omax_hbm.at[pl.ds(lo * C, R * C)])
        pltpu.sync_copy(acc_sum, omean_hbm.at[pl.ds(lo * C, R * C)])

    return sc_kernel(src, dst, g, rb)


def kernel(x_g, x_rsc, edge_index, W, b):
    src = edge_index[0]
    dst = edge_index[1]
    pad = ((0, NPAD - N), (0, 0))
    xg = jnp.pad(x_g, pad)
    xr = jnp.pad(x_rsc, pad)
    g, rb = _tc_tables(xg, xr, W, b)
    omax, omean = _sc_segment_reduce(src, dst, g, rb)
    omax = omax.reshape(NPAD, C)[:N]
    omean = omean.reshape(NPAD, C)[:N]
    return jnp.concatenate([omax, omean], axis=-1)


# speedup vs baseline: 1.1593x; 1.0116x over previous
import dataclasses
import functools

import jax
import jax.numpy as jnp
from jax import lax
from jax.experimental import pallas as pl
from jax.experimental.pallas import tpu as pltpu
from jax.experimental.pallas import tpu_sc as plsc

N = 10000
E = 320000
C = 128

NW = 32
R = 320
NPAD = NW * R
CHUNK = 2000
GB = 64
CAP = 8192
MASK = CAP - 1
TH = CAP - CHUNK
L = 16


def _tc_tables(xg, xr, W, b):
    BLK = 1024
    dn = (((1,), (1,)), ((), ()))

    def body(xg_ref, xr_ref, w_ref, b_ref, g_ref, rb_ref):
        w = w_ref[...]
        g_ref[...] = lax.dot_general(xg_ref[...], w, dn,
                                     preferred_element_type=jnp.float32)
        rb_ref[...] = lax.dot_general(xr_ref[...], w, dn,
                                      preferred_element_type=jnp.float32) - b_ref[...]

    return pl.pallas_call(
        body,
        grid=(NPAD // BLK,),
        in_specs=[
            pl.BlockSpec((BLK, C), lambda i: (i, 0)),
            pl.BlockSpec((BLK, C), lambda i: (i, 0)),
            pl.BlockSpec((C, C), lambda i: (0, 0)),
            pl.BlockSpec((1, C), lambda i: (0, 0)),
        ],
        out_specs=[
            pl.BlockSpec((BLK, C), lambda i: (i, 0)),
            pl.BlockSpec((BLK, C), lambda i: (i, 0)),
        ],
        out_shape=[jax.ShapeDtypeStruct((NPAD, C), jnp.float32)] * 2,
    )(xg, xr, W, b.reshape(1, C))


def _sc_segment_reduce(src, dst, g, rb):
    mesh = plsc.VectorSubcoreMesh(core_axis_name="c", subcore_axis_name="s")
    cp = pltpu.CompilerParams()
    if "needs_layout_passes" in pltpu.CompilerParams.__dataclass_fields__:
        cp = dataclasses.replace(cp, needs_layout_passes=False)

    @functools.partial(
        pl.kernel,
        out_type=[jax.ShapeDtypeStruct((NPAD * C,), jnp.float32)] * 2,
        mesh=mesh,
        compiler_params=cp,
        scratch_types=[
            pltpu.VMEM((R * C,), jnp.float32),
            pltpu.VMEM((R * C,), jnp.float32),
            pltpu.VMEM((R * L,), jnp.float32),
            pltpu.VMEM((CHUNK,), jnp.int32),
            pltpu.VMEM((CHUNK,), jnp.int32),
            pltpu.VMEM((CHUNK,), jnp.int32),
            pltpu.VMEM((CHUNK,), jnp.int32),
            pltpu.VMEM((CAP,), jnp.int32),
            pltpu.VMEM((CAP,), jnp.int32),
            pltpu.VMEM((GB, C), jnp.float32),
            pltpu.VMEM((GB, C), jnp.float32),
            pltpu.SemaphoreType.DMA,
            pltpu.SemaphoreType.DMA,
            pltpu.SemaphoreType.DMA,
            pltpu.SemaphoreType.DMA,
        ],
    )
    def sc_kernel(src_hbm, dst_hbm, g_hbm, rb_hbm, omax_hbm, omean_hbm,
                  acc_max, acc_sum, cnt, sbufa, dbufa, sbufb, dbufb,
                  slist, dlist, gr0, rr0,
                  semca, semcb, semg0, semr0):
        wid = lax.axis_index("s") * 2 + lax.axis_index("c")
        lo = wid * R
        zvec = jnp.zeros((L,), jnp.float32)
        zivec = jnp.zeros((L,), jnp.int32)
        ones = jnp.ones((L,), jnp.float32)
        lanes = lax.iota(jnp.int32, L)

        @pl.loop(0, R * C, step=L)
        def _init(i):
            acc_max[pl.ds(i, L)] = zvec
            acc_sum[pl.ds(i, L)] = zvec

        @pl.loop(0, R * L, step=L)
        def _init_cnt(i):
            cnt[pl.ds(i, L)] = zvec

        @pl.loop(0, CAP, step=L)
        def _init_lists(i):
            slist[pl.ds(i, L)] = zivec
            dlist[pl.ds(i, L)] = zivec

        def fire_gather(b0, grows, rrows, sg, sr):
            cg = pltpu.make_async_copy(g_hbm.at[slist.at[pl.ds(b0, GB)]],
                                       grows, sg)
            cr = pltpu.make_async_copy(rb_hbm.at[dlist.at[pl.ds(b0, GB)]],
                                       rrows, sr)
            cg.start()
            cr.start()

        def wait_gather(b0, grows, rrows, sg, sr):
            pltpu.make_async_copy(g_hbm.at[slist.at[pl.ds(b0, GB)]],
                                  grows, sg).wait()
            pltpu.make_async_copy(rb_hbm.at[dlist.at[pl.ds(b0, GB)]],
                                  rrows, sr).wait()

        def accum(b0, nb, grows, rrows):
            def edge(j, _):
                pos = b0 + j
                dvj = plsc.load_gather(dlist, [jnp.full((L,), pos, jnp.int32)])
                rowbase = (dvj - lo) * C + lanes
                for c in range(0, C, L):
                    gv = grows[j, pl.ds(c, L)]
                    rv = rrows[j, pl.ds(c, L)]
                    z = jnp.maximum(gv - rv, 0.0)
                    idxv = rowbase + c
                    mold = plsc.load_gather(acc_max, [idxv])
                    plsc.store_scatter(acc_max, [idxv], jnp.maximum(mold, z))
                    plsc.addupdate_scatter(acc_sum, [idxv], z)
                plsc.addupdate_scatter(cnt, [(dvj - lo) * L + lanes], ones)
                return 0

            lax.fori_loop(0, nb, edge, 0,
                          unroll=2 if isinstance(nb, int) else 1)

        def fire_pair(b0):
            fire_gather(b0, gr0, rr0, semg0, semr0)

        def consume_pair(b0, n0):
            wait_gather(b0, gr0, rr0, semg0, semr0)
            accum(b0, n0, gr0, rr0)

        def drain_pairs(cons, off, floor):
            def cond(cn):
                return off - cn > floor

            def body(cn):
                b0 = pl.multiple_of(cn & MASK, GB)
                fire_pair(b0)
                consume_pair(b0, GB)
                return cn + GB

            return lax.while_loop(cond, body, cons)

        def fire_chunk(ci, sbuf, dbuf, sem):
            e0 = ci * CHUNK
            pltpu.make_async_copy(src_hbm.at[pl.ds(e0, CHUNK)], sbuf,
                                  sem).start()
            pltpu.make_async_copy(dst_hbm.at[pl.ds(e0, CHUNK)], dbuf,
                                  sem).start()

        def wait_chunk(ci, sbuf, dbuf, sem):
            e0 = ci * CHUNK
            pltpu.make_async_copy(src_hbm.at[pl.ds(e0, CHUNK)], sbuf,
                                  sem).wait()
            pltpu.make_async_copy(dst_hbm.at[pl.ds(e0, CHUNK)], dbuf,
                                  sem).wait()

        def scan_chunk(sbuf, dbuf, off_v):
            def body(k, ov):
                d = dbuf[pl.ds(k * L, L)]
                s = sbuf[pl.ds(k * L, L)]
                ui = d - lo
                m = plsc.bitcast(ui, jnp.uint32) < jnp.uint32(R)
                mi = jnp.where(m, 1, 0)
                pos = (ov + plsc.cumsum(mi) - mi) & MASK
                plsc.store_scatter(slist, [pos], s, mask=m)
                plsc.store_scatter(dlist, [pos], d, mask=m)
                return ov + plsc.all_reduce_population_count(m)

            return lax.fori_loop(0, CHUNK // L, body, off_v, unroll=4)

        NCH = E // CHUNK
        fire_chunk(0, sbufa, dbufa, semca)

        def half_step(off_v, cons):
            off = jnp.max(off_v)
            do = off - cons >= GB
            b0 = pl.multiple_of(cons & MASK, GB)

            @pl.when(do)
            def _():
                fire_pair(b0)

            return off, do, b0

        def half_finish(off_v, do, b0, cons):
            @pl.when(do)
            def _():
                consume_pair(b0, GB)

            cons = jnp.where(do, cons + GB, cons)
            return drain_pairs(cons, jnp.max(off_v), TH)

        def chunk_pair(t, carry):
            off_v, cons = carry
            ci = 2 * t
            wait_chunk(ci, sbufa, dbufa, semca)
            fire_chunk(ci + 1, sbufb, dbufb, semcb)
            off, do, b0 = half_step(off_v, cons)
            off_v = scan_chunk(sbufa, dbufa, off_v)
            cons = half_finish(off_v, do, b0, cons)
            wait_chunk(ci + 1, sbufb, dbufb, semcb)
            fire_chunk(jnp.minimum(ci + 2, NCH - 2), sbufa, dbufa, semca)
            off, do, b0 = half_step(off_v, cons)
            off_v = scan_chunk(sbufb, dbufb, off_v)
            cons = half_finish(off_v, do, b0, cons)
            return off_v, cons

        off_v, cons = lax.fori_loop(
            0, NCH // 2, chunk_pair,
            (jnp.zeros((L,), jnp.int32), jnp.int32(0)))
        wait_chunk(NCH - 2, sbufa, dbufa, semca)
        off = jnp.max(off_v)
        cons = drain_pairs(cons, off, GB - 1)
        pend = off - cons
        bf = pl.multiple_of(cons & MASK, GB)
        fire_pair(bf)
        consume_pair(bf, jnp.maximum(0, jnp.minimum(GB, pend)))

        @pl.loop(0, R)
        def _finalize(r):
            denom = jnp.maximum(cnt[pl.ds(r * L, L)], 1.0)
            for c in range(0, C, L):
                slc = acc_sum.at[pl.ds(r * C + c, L)]
                slc[...] = slc[...] / denom

        pltpu.sync_copy(acc_max, omax_hbm.at[pl.ds(lo * C, R * C)])
        pltpu.sync_copy(acc_sum, omean_hbm.at[pl.ds(lo * C, R * C)])

    return sc_kernel(src, dst, g, rb)


def kernel(x_g, x_rsc, edge_index, W, b):
    src = edge_index[0]
    dst = edge_index[1]
    pad = ((0, NPAD - N), (0, 0))
    xg = jnp.pad(x_g, pad)
    xr = jnp.pad(x_rsc, pad)
    g, rb = _tc_tables(xg, xr, W, b)
    omax, omean = _sc_segment_reduce(src, dst, g, rb)
    omax = omax.reshape(NPAD, C)[:N]
    omean = omean.reshape(NPAD, C)[:N]
    return jnp.concatenate([omax, omean], axis=-1)
